# baseline (speedup 1.0000x reference)
import functools

import jax
import jax.numpy as jnp
from jax import lax
from jax.experimental import pallas as pl
from jax.experimental.pallas import tpu as pltpu
from jax.experimental.pallas import tpu_sc as plsc

N = 10000
E = 320000
EP = 100000
D = 128
H = 128
HH = H // 2

NC = 2
NS = 16
NW = NC * NS
L = 16

NP = 10240
RPT = NP // NS

CH = 256
HCH = 40
NCH_E = 2 * HCH
EPAD = NS * NCH_E * CH
KR = 3
GA = 1

CHD = 64
MZ_RAW = 2 * EP
NCH_Z = -(-MZ_RAW // (NW * CHD))
MZ = NW * NCH_Z * CHD
KD = 4
GD = 3

_MESH = dict(core_axis_name="c", subcore_axis_name="s", num_cores=NC,
             num_subcores=NS)


def _make_sc_aggregate(with_deg):
  out_type = (jax.ShapeDtypeStruct((NC, NP, HH), jnp.float32),)
  if with_deg:
    out_type += (jax.ShapeDtypeStruct((NP,), jnp.float32),)

  @functools.partial(
      pl.kernel,
      out_type=out_type,
      mesh=plsc.VectorSubcoreMesh(**_MESH),
      scratch_types=[
          pltpu.VMEM((HCH, CH), jnp.int32),
          pltpu.VMEM((HCH, CH), jnp.int32),
          pltpu.VMEM((KR, CH, HH), jnp.float32),
          pltpu.VMEM((CH,), jnp.float32),
          pltpu.VMEM((CH,), jnp.float32),
          pltpu.VMEM_SHARED((NP, HH), jnp.float32),
          pltpu.VMEM_SHARED((NP,), jnp.float32),
          pltpu.SemaphoreType.DMA,
          pltpu.SemaphoreType.DMA,
          pltpu.SemaphoreType.DMA,
      ],
      compiler_params=pltpu.CompilerParams(use_tc_tiling_on_sc=False),
  )
  def _sc_aggregate(y_hbm, src_hbm, dst_hbm, *out_and_scratch):
    if with_deg:
        (agg_out, deg_out, sidx_v, didx_v, rows_v, ones_v, zeros_v,
         acc_sh, deg_sh, gsem, ssem, dsem) = out_and_scratch
    else:
        (agg_out, sidx_v, didx_v, rows_v, ones_v, zeros_v,
         acc_sh, deg_sh, gsem, ssem, dsem) = out_and_scratch
    cid = lax.axis_index("c")
    sid = lax.axis_index("s")
    pltpu.sync_copy(src_hbm.at[sid, 0], sidx_v)
    pltpu.sync_copy(dst_hbm.at[sid, 0], didx_v)

    zeros = jnp.zeros((L,), jnp.float32)
    ones = jnp.ones((L,), jnp.float32)
    for i in range(CH // L):
        ones_v[pl.ds(i * L, L)] = ones
        zeros_v[pl.ds(i * L, L)] = zeros

    def _zrow(r, _):
        for c in range(HH // L):
            rows_v[0, r, pl.ds(c * L, L)] = zeros
        return 0
    lax.fori_loop(0, CH, _zrow, 0)
    for k in range(RPT // CH):
        pltpu.sync_copy(rows_v.at[0],
                        acc_sh.at[pl.ds(sid * RPT + k * CH, CH)])
    if RPT % CH:
        pltpu.sync_copy(rows_v.at[0, pl.ds(0, RPT % CH)],
                        acc_sh.at[pl.ds(sid * RPT + (RPT // CH) * CH,
                                        RPT % CH)])
    if with_deg:
        for k in range(RPT // 128):
            pltpu.sync_copy(zeros_v.at[pl.ds(0, 128)],
                            deg_sh.at[pl.ds(sid * RPT + k * 128, 128)])
    plsc.subcore_barrier()

    def _run(y_c, do_deg):
        for g in range(GA):
            pltpu.async_copy(y_c.at[sidx_v.at[g]], rows_v.at[g], gsem)

        def _step(j, _):
            s = lax.rem(j, KR)
            p = lax.rem(j + GA, KR)
            jr = lax.rem(j, HCH)

            @pl.when(j + GA >= KR)
            def _():
                pltpu.make_async_copy(rows_v.at[0], acc_sh.at[didx_v.at[0]],
                                      ssem).wait()

            pltpu.make_async_copy(y_c.at[sidx_v.at[0]], rows_v.at[0],
                                  gsem).wait()

            @pl.when(j == HCH - 1)
            def _():
                pltpu.sync_copy(src_hbm.at[sid, 1], sidx_v)

            @pl.when(j + GA < NCH_E)
            def _():
                pltpu.async_copy(y_c.at[sidx_v.at[lax.rem(j + GA, HCH)]],
                                 rows_v.at[p], gsem)

            pltpu.async_copy(rows_v.at[s], acc_sh.at[didx_v.at[jr]], ssem,
                             add=True)
            if do_deg:
                @pl.when(j >= 1)
                def _():
                    pltpu.make_async_copy(ones_v, deg_sh.at[didx_v.at[0]],
                                          dsem).wait()
                pltpu.async_copy(ones_v, deg_sh.at[didx_v.at[jr]], dsem,
                                 add=True)

            @pl.when(j == HCH - 1)
            def _():
                pltpu.sync_copy(dst_hbm.at[sid, 1], didx_v)
            return 0

        lax.fori_loop(0, NCH_E, _step, 0)
        for _ in range(KR - GA):
            pltpu.make_async_copy(rows_v.at[0], acc_sh.at[didx_v.at[0]],
                                  ssem).wait()
        if do_deg:
            pltpu.make_async_copy(ones_v, deg_sh.at[didx_v.at[0]],
                                  dsem).wait()

    @pl.when(cid == 0)
    def _():
        _run(y_hbm.at[0], with_deg)

    @pl.when(cid == 1)
    def _():
        _run(y_hbm.at[1], False)

    plsc.subcore_barrier()

    for k in range(RPT // 128):
        r0 = sid * RPT + k * 128
        pltpu.sync_copy(acc_sh.at[pl.ds(r0, 128)],
                        agg_out.at[cid, pl.ds(r0, 128)])

    if with_deg:
        @pl.when(cid == 0)
        def _():
            pltpu.sync_copy(deg_sh.at[pl.ds(sid * RPT, RPT)],
                            deg_out.at[pl.ds(sid * RPT, RPT)])

  return _sc_aggregate


_sc_aggregate_deg = _make_sc_aggregate(True)
_sc_aggregate_nodeg = _make_sc_aggregate(False)


@functools.partial(
    pl.kernel,
    out_type=jax.ShapeDtypeStruct((MZ, H), jnp.float32),
    mesh=plsc.VectorSubcoreMesh(**_MESH),
    scratch_types=[
        pltpu.VMEM((NCH_Z, CHD), jnp.int32),
        pltpu.VMEM((NCH_Z, CHD), jnp.int32),
        pltpu.VMEM((KD, CHD, H), jnp.float32),
        pltpu.VMEM((KD, CHD, H), jnp.float32),
        pltpu.VMEM((2, CHD, H), jnp.float32),
        pltpu.SemaphoreType.DMA,
        pltpu.SemaphoreType.DMA,
    ],
)
def _sc_decode_gather(h_hbm, aidx_hbm, bidx_hbm, z_out,
                      aidx_v, bidx_v, ra_v, rb_v, rz_v, gsem, wsem):
    cid = lax.axis_index("c")
    sid = lax.axis_index("s")
    wid = sid * NC + cid
    pltpu.sync_copy(aidx_hbm.at[wid], aidx_v)
    pltpu.sync_copy(bidx_hbm.at[wid], bidx_v)

    base = wid * NCH_Z * CHD

    for g in range(GD):
        pltpu.async_copy(h_hbm.at[aidx_v.at[g]], ra_v.at[g], gsem)
        pltpu.async_copy(h_hbm.at[bidx_v.at[g]], rb_v.at[g], gsem)

    def _step(j, _):
        s = lax.rem(j, KD)
        p = lax.rem(j + GD, KD)
        zs = lax.rem(j, 2)
        pltpu.make_async_copy(h_hbm.at[aidx_v.at[0]], ra_v.at[0], gsem).wait()
        pltpu.make_async_copy(h_hbm.at[bidx_v.at[0]], rb_v.at[0], gsem).wait()

        @pl.when(j + GD < NCH_Z)
        def _():
            pltpu.async_copy(h_hbm.at[aidx_v.at[j + GD]], ra_v.at[p], gsem)
            pltpu.async_copy(h_hbm.at[bidx_v.at[j + GD]], rb_v.at[p], gsem)

        @pl.when(j >= 2)
        def _():
            pltpu.make_async_copy(rz_v.at[0], z_out.at[pl.ds(0, CHD)],
                                  wsem).wait()

        def _mulrow(r2, _):
            for rr in range(2):
                r = r2 * 2 + rr
                for c in range(H // L):
                    sl = pl.ds(c * L, L)
                    rz_v[zs, r, sl] = ra_v[s, r, sl] * rb_v[s, r, sl]
            return 0
        lax.fori_loop(0, CHD // 2, _mulrow, 0)
        pltpu.async_copy(rz_v.at[zs], z_out.at[pl.ds(base + j * CHD, CHD)],
                         wsem)
        return 0

    lax.fori_loop(0, NCH_Z, _step, 0)
    pltpu.make_async_copy(rz_v.at[0], z_out.at[pl.ds(0, CHD)], wsem).wait()
    pltpu.make_async_copy(rz_v.at[0], z_out.at[pl.ds(0, CHD)], wsem).wait()


_BLK = 1024


def _tc_enc1_body(x_ref, ws_ref, wn_ref, b_ref, xs_ref, y_ref):
    xb = x_ref[...]
    xs_ref[...] = jnp.dot(xb, ws_ref[...],
                          preferred_element_type=jnp.float32) + b_ref[...]
    y = jnp.dot(xb, wn_ref[...], preferred_element_type=jnp.float32)
    y_ref[0] = y[:, :HH]
    y_ref[1] = y[:, HH:]


def _tc_enc1(xp, Ws1, Wn1, b1):
    return pl.pallas_call(
        _tc_enc1_body,
        grid=(NP // _BLK,),
        in_specs=[
            pl.BlockSpec((_BLK, D), lambda i: (i, 0)),
            pl.BlockSpec((D, H), lambda i: (0, 0)),
            pl.BlockSpec((D, H), lambda i: (0, 0)),
            pl.BlockSpec((1, H), lambda i: (0, 0)),
        ],
        out_specs=[
            pl.BlockSpec((_BLK, H), lambda i: (i, 0)),
            pl.BlockSpec((NC, _BLK, HH), lambda i: (0, i, 0)),
        ],
        out_shape=[
            jax.ShapeDtypeStruct((NP, H), jnp.float32),
            jax.ShapeDtypeStruct((NC, NP, HH), jnp.float32),
        ],
    )(xp, Ws1, Wn1, b1.reshape(1, H))


def _mean_from_parts(agg_ref, deg_ref):
    agg = jnp.concatenate([agg_ref[0], agg_ref[1]], axis=1)
    return agg * (1.0 / jnp.maximum(deg_ref[...], 1.0))


def _tc_enc2_body(xs_ref, agg_ref, deg_ref, ws_ref, wn_ref, b_ref,
                  hs_ref, y_ref):
    h1 = jnp.maximum(xs_ref[...] + _mean_from_parts(agg_ref, deg_ref), 0.0)
    hs_ref[...] = jnp.dot(h1, ws_ref[...],
                          preferred_element_type=jnp.float32) + b_ref[...]
    y = jnp.dot(h1, wn_ref[...], preferred_element_type=jnp.float32)
    y_ref[0] = y[:, :HH]
    y_ref[1] = y[:, HH:]


def _tc_enc2(xs1, agg1, deg1, Ws2, Wn2, b2):
    return pl.pallas_call(
        _tc_enc2_body,
        grid=(NP // _BLK,),
        in_specs=[
            pl.BlockSpec((_BLK, H), lambda i: (i, 0)),
            pl.BlockSpec((NC, _BLK, HH), lambda i: (0, i, 0)),
            pl.BlockSpec((_BLK, 1), lambda i: (i, 0)),
            pl.BlockSpec((H, H), lambda i: (0, 0)),
            pl.BlockSpec((H, H), lambda i: (0, 0)),
            pl.BlockSpec((1, H), lambda i: (0, 0)),
        ],
        out_specs=[
            pl.BlockSpec((_BLK, H), lambda i: (i, 0)),
            pl.BlockSpec((NC, _BLK, HH), lambda i: (0, i, 0)),
        ],
        out_shape=[
            jax.ShapeDtypeStruct((NP, H), jnp.float32),
            jax.ShapeDtypeStruct((NC, NP, HH), jnp.float32),
        ],
    )(xs1, agg1, deg1.reshape(NP, 1), Ws2, Wn2, b2.reshape(1, H))


def _tc_combine_body(hs_ref, agg_ref, deg_ref, h_ref):
    h_ref[...] = hs_ref[...] + _mean_from_parts(agg_ref, deg_ref)


def _tc_combine(hs2, agg2, deg1):
    return pl.pallas_call(
        _tc_combine_body,
        grid=(NP // _BLK,),
        in_specs=[
            pl.BlockSpec((_BLK, H), lambda i: (i, 0)),
            pl.BlockSpec((NC, _BLK, HH), lambda i: (0, i, 0)),
            pl.BlockSpec((_BLK, 1), lambda i: (i, 0)),
        ],
        out_specs=pl.BlockSpec((_BLK, H), lambda i: (i, 0)),
        out_shape=jax.ShapeDtypeStruct((NP, H), jnp.float32),
    )(hs2, agg2, deg1.reshape(NP, 1))


_BLKZ = 2048


def _tc_mlp_body(z_ref, w1_ref, b1_ref, w2_ref, b2_ref, w3_ref, b3_ref,
                 o_ref):
    z = z_ref[...]
    z = jnp.maximum(jnp.dot(z, w1_ref[...],
                            preferred_element_type=jnp.float32)
                    + b1_ref[...], 0.0)
    z = jnp.maximum(jnp.dot(z, w2_ref[...],
                            preferred_element_type=jnp.float32)
                    + b2_ref[...], 0.0)
    o_ref[...] = jnp.dot(z, w3_ref[...],
                         preferred_element_type=jnp.float32) + b3_ref[...]


def _tc_mlp(z, Wd1, bd1, Wd2, bd2, Wd3, bd3):
    return pl.pallas_call(
        _tc_mlp_body,
        grid=(MZ // _BLKZ,),
        in_specs=[
            pl.BlockSpec((_BLKZ, H), lambda i: (i, 0)),
            pl.BlockSpec((H, H), lambda i: (0, 0)),
            pl.BlockSpec((1, H), lambda i: (0, 0)),
            pl.BlockSpec((H, H), lambda i: (0, 0)),
            pl.BlockSpec((1, H), lambda i: (0, 0)),
            pl.BlockSpec((H, 1), lambda i: (0, 0)),
            pl.BlockSpec((1, 1), lambda i: (0, 0)),
        ],
        out_specs=pl.BlockSpec((_BLKZ, 1), lambda i: (i, 0)),
        out_shape=jax.ShapeDtypeStruct((MZ, 1), jnp.float32),
    )(z, Wd1, bd1.reshape(1, H), Wd2, bd2.reshape(1, H), Wd3,
      bd3.reshape(1, 1))


def kernel(x, edge_index, pos_edge_index, neg_edge_index, Ws1, Wn1, b1,
           Ws2, Wn2, b2, Wd1, bd1, Wd2, bd2, Wd3, bd3):
    xp = jnp.pad(x, ((0, NP - N), (0, 0)))

    pad_e = jnp.full((EPAD - E,), NP - 1, jnp.int32)
    srcp = jnp.concatenate([edge_index[0], pad_e]).reshape(NS, 2, HCH, CH)
    dstp = jnp.concatenate([edge_index[1], pad_e]).reshape(NS, 2, HCH, CH)

    pad_z = jnp.zeros((MZ - MZ_RAW,), jnp.int32)
    aidx = jnp.concatenate(
        [pos_edge_index[0], neg_edge_index[0], pad_z]).reshape(NW, NCH_Z, CHD)
    bidx = jnp.concatenate(
        [pos_edge_index[1], neg_edge_index[1], pad_z]).reshape(NW, NCH_Z, CHD)
    srcp, dstp, aidx, bidx = lax.optimization_barrier((srcp, dstp, aidx, bidx))

    xs1, y1 = _tc_enc1(xp, Ws1, Wn1, b1)
    agg1, deg1 = _sc_aggregate_deg(y1, srcp, dstp)
    hs2, y2 = _tc_enc2(xs1, agg1, deg1, Ws2, Wn2, b2)
    agg2, = _sc_aggregate_nodeg(y2, srcp, dstp)
    h = _tc_combine(hs2, agg2, deg1)
    z = _sc_decode_gather(h, aidx, bidx)
    scores = _tc_mlp(z, Wd1, bd1, Wd2, bd2, Wd3, bd3)
    return (scores[:EP], scores[EP:2 * EP])


# speedup vs baseline: 1.0936x; 1.0936x over previous
import functools

import jax
import jax.numpy as jnp
from jax import lax
from jax.experimental import pallas as pl
from jax.experimental.pallas import tpu as pltpu
from jax.experimental.pallas import tpu_sc as plsc

N = 10000
E = 320000
EP = 100000
D = 128
H = 128
HH = H // 2

NC = 2
NS = 16
NW = NC * NS
L = 16

NP = 10240
RPT = NP // NS

CH = 128
NCH_E = -(-E // (NS * CH))
NCH_E = -(-NCH_E // 8) * 8
EPAD = NS * NCH_E * CH
KR = 5
GA = 2

CHD = 64
MZ_RAW = 2 * EP
NCH_Z = -(-MZ_RAW // (NW * CHD))
MZ = NW * NCH_Z * CHD
KD = 4
GD2 = 2

_MESH = dict(core_axis_name="c", subcore_axis_name="s", num_cores=NC,
             num_subcores=NS)


def _make_sc_aggregate(with_deg):
  out_type = (jax.ShapeDtypeStruct((NC, NP, HH), jnp.float32),)
  if with_deg:
    out_type += (jax.ShapeDtypeStruct((NP,), jnp.float32),)

  @functools.partial(
      pl.kernel,
      out_type=out_type,
      mesh=plsc.VectorSubcoreMesh(**_MESH),
      scratch_types=[
          pltpu.VMEM((NCH_E, CH), jnp.int32),
          pltpu.VMEM((NCH_E, CH), jnp.int32),
          pltpu.VMEM((KR, CH, HH), jnp.float32),
          pltpu.VMEM((CH,), jnp.float32),
          pltpu.VMEM((CH,), jnp.float32),
          pltpu.VMEM_SHARED((NP, HH), jnp.float32),
          pltpu.VMEM_SHARED((NP,), jnp.float32),
          pltpu.SemaphoreType.DMA,
          pltpu.SemaphoreType.DMA,
          pltpu.SemaphoreType.DMA,
      ],
      compiler_params=pltpu.CompilerParams(use_tc_tiling_on_sc=False),
  )
  def _sc_aggregate(y_hbm, src_hbm, dst_hbm, *out_and_scratch):
    if with_deg:
        (agg_out, deg_out, sidx_v, didx_v, rows_v, ones_v, zeros_v,
         acc_sh, deg_sh, gsem, ssem, dsem) = out_and_scratch
    else:
        (agg_out, sidx_v, didx_v, rows_v, ones_v, zeros_v,
         acc_sh, deg_sh, gsem, ssem, dsem) = out_and_scratch
    cid = lax.axis_index("c")
    sid = lax.axis_index("s")
    pltpu.sync_copy(src_hbm.at[sid], sidx_v)
    pltpu.sync_copy(dst_hbm.at[sid], didx_v)

    zeros = jnp.zeros((L,), jnp.float32)
    ones = jnp.ones((L,), jnp.float32)
    for i in range(CH // L):
        ones_v[pl.ds(i * L, L)] = ones
        zeros_v[pl.ds(i * L, L)] = zeros

    def _zrow(r, _):
        for c in range(HH // L):
            rows_v[0, r, pl.ds(c * L, L)] = zeros
        return 0
    lax.fori_loop(0, CH, _zrow, 0)
    for k in range(RPT // CH):
        pltpu.sync_copy(rows_v.at[0],
                        acc_sh.at[pl.ds(sid * RPT + k * CH, CH)])
        if with_deg:
            pltpu.sync_copy(zeros_v,
                            deg_sh.at[pl.ds(sid * RPT + k * CH, CH)])
    plsc.subcore_barrier()

    def _run(y_c, do_deg):
        for g in range(GA):
            pltpu.async_copy(y_c.at[sidx_v.at[g]], rows_v.at[g], gsem)

        def _step(j, _):
            s = lax.rem(j, KR)
            p = lax.rem(j + GA, KR)

            @pl.when(j + GA >= KR)
            def _():
                pltpu.make_async_copy(rows_v.at[0], acc_sh.at[didx_v.at[0]],
                                      ssem).wait()

            pltpu.make_async_copy(y_c.at[sidx_v.at[0]], rows_v.at[0],
                                  gsem).wait()

            @pl.when(j + GA < NCH_E)
            def _():
                pltpu.async_copy(y_c.at[sidx_v.at[j + GA]], rows_v.at[p],
                                 gsem)

            pltpu.async_copy(rows_v.at[s], acc_sh.at[didx_v.at[j]], ssem,
                             add=True)
            if do_deg:
                @pl.when(j >= 1)
                def _():
                    pltpu.make_async_copy(ones_v, deg_sh.at[didx_v.at[0]],
                                          dsem).wait()
                pltpu.async_copy(ones_v, deg_sh.at[didx_v.at[j]], dsem,
                                 add=True)
            return 0

        lax.fori_loop(0, NCH_E, _step, 0)
        for _ in range(KR - GA):
            pltpu.make_async_copy(rows_v.at[0], acc_sh.at[didx_v.at[0]],
                                  ssem).wait()
        if do_deg:
            pltpu.make_async_copy(ones_v, deg_sh.at[didx_v.at[0]],
                                  dsem).wait()

    @pl.when(cid == 0)
    def _():
        _run(y_hbm.at[0], with_deg)

    @pl.when(cid == 1)
    def _():
        _run(y_hbm.at[1], False)

    plsc.subcore_barrier()

    for k in range(RPT // CH):
        r0 = sid * RPT + k * CH
        pltpu.sync_copy(acc_sh.at[pl.ds(r0, CH)],
                        agg_out.at[cid, pl.ds(r0, CH)])

    if with_deg:
        @pl.when(cid == 0)
        def _():
            pltpu.sync_copy(deg_sh.at[pl.ds(sid * RPT, RPT)],
                            deg_out.at[pl.ds(sid * RPT, RPT)])

  return _sc_aggregate


_sc_aggregate_deg = _make_sc_aggregate(True)
_sc_aggregate_nodeg = _make_sc_aggregate(False)


@functools.partial(
    pl.kernel,
    out_type=jax.ShapeDtypeStruct((2, MZ, H), jnp.float32),
    mesh=plsc.VectorSubcoreMesh(**_MESH),
    scratch_types=[
        pltpu.VMEM((NCH_Z, CHD), jnp.int32),
        pltpu.VMEM((NCH_Z, CHD), jnp.int32),
        pltpu.VMEM((KD, CHD, H), jnp.float32),
        pltpu.VMEM((KD, CHD, H), jnp.float32),
        pltpu.SemaphoreType.DMA,
        pltpu.SemaphoreType.DMA,
    ],
)
def _sc_decode_gather(h_hbm, aidx_hbm, bidx_hbm, z_out,
                      aidx_v, bidx_v, ra_v, rb_v, gsem, wsem):
    cid = lax.axis_index("c")
    sid = lax.axis_index("s")
    wid = sid * NC + cid
    pltpu.sync_copy(aidx_hbm.at[wid], aidx_v)
    pltpu.sync_copy(bidx_hbm.at[wid], bidx_v)

    base = wid * NCH_Z * CHD

    for g in range(GD2):
        pltpu.async_copy(h_hbm.at[aidx_v.at[g]], ra_v.at[g], gsem)
        pltpu.async_copy(h_hbm.at[bidx_v.at[g]], rb_v.at[g], gsem)

    def _step(j, _):
        s = lax.rem(j, KD)
        p = lax.rem(j + GD2, KD)

        @pl.when(j + GD2 >= KD)
        def _():
            pltpu.make_async_copy(ra_v.at[0], z_out.at[0, pl.ds(0, CHD)],
                                  wsem).wait()
            pltpu.make_async_copy(ra_v.at[0], z_out.at[0, pl.ds(0, CHD)],
                                  wsem).wait()

        pltpu.make_async_copy(h_hbm.at[aidx_v.at[0]], ra_v.at[0], gsem).wait()
        pltpu.make_async_copy(h_hbm.at[bidx_v.at[0]], rb_v.at[0], gsem).wait()

        @pl.when(j + GD2 < NCH_Z)
        def _():
            pltpu.async_copy(h_hbm.at[aidx_v.at[j + GD2]], ra_v.at[p], gsem)
            pltpu.async_copy(h_hbm.at[bidx_v.at[j + GD2]], rb_v.at[p], gsem)

        pltpu.async_copy(ra_v.at[s], z_out.at[0, pl.ds(base + j * CHD, CHD)],
                         wsem)
        pltpu.async_copy(rb_v.at[s], z_out.at[1, pl.ds(base + j * CHD, CHD)],
                         wsem)
        return 0

    lax.fori_loop(0, NCH_Z, _step, 0)
    for _ in range(2 * (KD - GD2)):
        pltpu.make_async_copy(ra_v.at[0], z_out.at[0, pl.ds(0, CHD)],
                              wsem).wait()


_BLK = 1024


def _tc_enc1_body(x_ref, ws_ref, wn_ref, b_ref, xs_ref, y_ref):
    xb = x_ref[...]
    xs_ref[...] = jnp.dot(xb, ws_ref[...],
                          preferred_element_type=jnp.float32) + b_ref[...]
    y = jnp.dot(xb, wn_ref[...], preferred_element_type=jnp.float32)
    y_ref[0] = y[:, :HH]
    y_ref[1] = y[:, HH:]


def _tc_enc1(xp, Ws1, Wn1, b1):
    return pl.pallas_call(
        _tc_enc1_body,
        grid=(NP // _BLK,),
        in_specs=[
            pl.BlockSpec((_BLK, D), lambda i: (i, 0)),
            pl.BlockSpec((D, H), lambda i: (0, 0)),
            pl.BlockSpec((D, H), lambda i: (0, 0)),
            pl.BlockSpec((1, H), lambda i: (0, 0)),
        ],
        out_specs=[
            pl.BlockSpec((_BLK, H), lambda i: (i, 0)),
            pl.BlockSpec((NC, _BLK, HH), lambda i: (0, i, 0)),
        ],
        out_shape=[
            jax.ShapeDtypeStruct((NP, H), jnp.float32),
            jax.ShapeDtypeStruct((NC, NP, HH), jnp.float32),
        ],
    )(xp, Ws1, Wn1, b1.reshape(1, H))


def _mean_from_parts(agg_ref, deg_ref):
    agg = jnp.concatenate([agg_ref[0], agg_ref[1]], axis=1)
    return agg * (1.0 / jnp.maximum(deg_ref[...], 1.0))


def _tc_enc2_body(xs_ref, agg_ref, deg_ref, ws_ref, wn_ref, b_ref,
                  hs_ref, y_ref):
    h1 = jnp.maximum(xs_ref[...] + _mean_from_parts(agg_ref, deg_ref), 0.0)
    hs_ref[...] = jnp.dot(h1, ws_ref[...],
                          preferred_element_type=jnp.float32) + b_ref[...]
    y = jnp.dot(h1, wn_ref[...], preferred_element_type=jnp.float32)
    y_ref[0] = y[:, :HH]
    y_ref[1] = y[:, HH:]


def _tc_enc2(xs1, agg1, deg1, Ws2, Wn2, b2):
    return pl.pallas_call(
        _tc_enc2_body,
        grid=(NP // _BLK,),
        in_specs=[
            pl.BlockSpec((_BLK, H), lambda i: (i, 0)),
            pl.BlockSpec((NC, _BLK, HH), lambda i: (0, i, 0)),
            pl.BlockSpec((_BLK, 1), lambda i: (i, 0)),
            pl.BlockSpec((H, H), lambda i: (0, 0)),
            pl.BlockSpec((H, H), lambda i: (0, 0)),
            pl.BlockSpec((1, H), lambda i: (0, 0)),
        ],
        out_specs=[
            pl.BlockSpec((_BLK, H), lambda i: (i, 0)),
            pl.BlockSpec((NC, _BLK, HH), lambda i: (0, i, 0)),
        ],
        out_shape=[
            jax.ShapeDtypeStruct((NP, H), jnp.float32),
            jax.ShapeDtypeStruct((NC, NP, HH), jnp.float32),
        ],
    )(xs1, agg1, deg1.reshape(NP, 1), Ws2, Wn2, b2.reshape(1, H))


def _tc_combine_body(hs_ref, agg_ref, deg_ref, h_ref):
    h_ref[...] = hs_ref[...] + _mean_from_parts(agg_ref, deg_ref)


def _tc_combine(hs2, agg2, deg1):
    return pl.pallas_call(
        _tc_combine_body,
        grid=(NP // _BLK,),
        in_specs=[
            pl.BlockSpec((_BLK, H), lambda i: (i, 0)),
            pl.BlockSpec((NC, _BLK, HH), lambda i: (0, i, 0)),
            pl.BlockSpec((_BLK, 1), lambda i: (i, 0)),
        ],
        out_specs=pl.BlockSpec((_BLK, H), lambda i: (i, 0)),
        out_shape=jax.ShapeDtypeStruct((NP, H), jnp.float32),
    )(hs2, agg2, deg1.reshape(NP, 1))


_BLKZ = 2048


def _tc_mlp_body(z_ref, w1_ref, b1_ref, w2_ref, b2_ref, w3_ref, b3_ref,
                 o_ref):
    z = z_ref[0] * z_ref[1]
    z = jnp.maximum(jnp.dot(z, w1_ref[...],
                            preferred_element_type=jnp.float32)
                    + b1_ref[...], 0.0)
    z = jnp.maximum(jnp.dot(z, w2_ref[...],
                            preferred_element_type=jnp.float32)
                    + b2_ref[...], 0.0)
    o_ref[...] = jnp.dot(z, w3_ref[...],
                         preferred_element_type=jnp.float32) + b3_ref[...]


def _tc_mlp(z, Wd1, bd1, Wd2, bd2, Wd3, bd3):
    return pl.pallas_call(
        _tc_mlp_body,
        grid=(MZ // _BLKZ,),
        in_specs=[
            pl.BlockSpec((2, _BLKZ, H), lambda i: (0, i, 0)),
            pl.BlockSpec((H, H), lambda i: (0, 0)),
            pl.BlockSpec((1, H), lambda i: (0, 0)),
            pl.BlockSpec((H, H), lambda i: (0, 0)),
            pl.BlockSpec((1, H), lambda i: (0, 0)),
            pl.BlockSpec((H, 1), lambda i: (0, 0)),
            pl.BlockSpec((1, 1), lambda i: (0, 0)),
        ],
        out_specs=pl.BlockSpec((_BLKZ, 1), lambda i: (i, 0)),
        out_shape=jax.ShapeDtypeStruct((MZ, 1), jnp.float32),
    )(z, Wd1, bd1.reshape(1, H), Wd2, bd2.reshape(1, H), Wd3,
      bd3.reshape(1, 1))


def kernel(x, edge_index, pos_edge_index, neg_edge_index, Ws1, Wn1, b1,
           Ws2, Wn2, b2, Wd1, bd1, Wd2, bd2, Wd3, bd3):
    xp = jnp.pad(x, ((0, NP - N), (0, 0)))

    pad_e = jnp.full((EPAD - E,), NP - 1, jnp.int32)
    srcp = jnp.concatenate([edge_index[0], pad_e]).reshape(NS, NCH_E, CH)
    dstp = jnp.concatenate([edge_index[1], pad_e]).reshape(NS, NCH_E, CH)

    pad_z = jnp.zeros((MZ - MZ_RAW,), jnp.int32)
    aidx = jnp.concatenate(
        [pos_edge_index[0], neg_edge_index[0], pad_z]).reshape(NW, NCH_Z, CHD)
    bidx = jnp.concatenate(
        [pos_edge_index[1], neg_edge_index[1], pad_z]).reshape(NW, NCH_Z, CHD)
    srcp, dstp, aidx, bidx = lax.optimization_barrier((srcp, dstp, aidx, bidx))

    xs1, y1 = _tc_enc1(xp, Ws1, Wn1, b1)
    agg1, deg1 = _sc_aggregate_deg(y1, srcp, dstp)
    hs2, y2 = _tc_enc2(xs1, agg1, deg1, Ws2, Wn2, b2)
    agg2, = _sc_aggregate_nodeg(y2, srcp, dstp)
    h = _tc_combine(hs2, agg2, deg1)
    z = _sc_decode_gather(h, aidx, bidx)
    scores = _tc_mlp(z, Wd1, bd1, Wd2, bd2, Wd3, bd3)
    return (scores[:EP], scores[EP:2 * EP])


# speedup vs baseline: 1.1024x; 1.0080x over previous
import functools

import jax
import jax.numpy as jnp
from jax import lax
from jax.experimental import pallas as pl
from jax.experimental.pallas import tpu as pltpu
from jax.experimental.pallas import tpu_sc as plsc

N = 10000
E = 320000
EP = 100000
D = 128
H = 128
HH = H // 2

NC = 2
NS = 16
NW = NC * NS
L = 16

NP = 10240
RPT = NP // NS

CH = 128
NCH_E = -(-E // (NS * CH))
NCH_E = -(-NCH_E // 8) * 8
EPAD = NS * NCH_E * CH
KR = 5
GA = 3

CHD = 64
MZ_RAW = 2 * EP
NCH_Z = -(-MZ_RAW // (NW * CHD))
MZ = NW * NCH_Z * CHD
KD = 6
GD2 = 4

_MESH = dict(core_axis_name="c", subcore_axis_name="s", num_cores=NC,
             num_subcores=NS)


def _make_sc_aggregate(with_deg):
  out_type = (jax.ShapeDtypeStruct((NC, NP, HH), jnp.float32),)
  if with_deg:
    out_type += (jax.ShapeDtypeStruct((NP,), jnp.float32),)

  @functools.partial(
      pl.kernel,
      out_type=out_type,
      mesh=plsc.VectorSubcoreMesh(**_MESH),
      scratch_types=[
          pltpu.VMEM((NCH_E, CH), jnp.int32),
          pltpu.VMEM((NCH_E, CH), jnp.int32),
          pltpu.VMEM((KR, CH, HH), jnp.float32),
          pltpu.VMEM((CH,), jnp.float32),
          pltpu.VMEM((CH,), jnp.float32),
          pltpu.VMEM_SHARED((NP, HH), jnp.float32),
          pltpu.VMEM_SHARED((NP,), jnp.float32),
          pltpu.SemaphoreType.DMA,
          pltpu.SemaphoreType.DMA,
          pltpu.SemaphoreType.DMA,
      ],
      compiler_params=pltpu.CompilerParams(use_tc_tiling_on_sc=False),
  )
  def _sc_aggregate(y_hbm, src_hbm, dst_hbm, *out_and_scratch):
    if with_deg:
        (agg_out, deg_out, sidx_v, didx_v, rows_v, ones_v, zeros_v,
         acc_sh, deg_sh, gsem, ssem, dsem) = out_and_scratch
    else:
        (agg_out, sidx_v, didx_v, rows_v, ones_v, zeros_v,
         acc_sh, deg_sh, gsem, ssem, dsem) = out_and_scratch
    cid = lax.axis_index("c")
    sid = lax.axis_index("s")
    pltpu.sync_copy(src_hbm.at[sid], sidx_v)
    pltpu.sync_copy(dst_hbm.at[sid], didx_v)

    zeros = jnp.zeros((L,), jnp.float32)
    ones = jnp.ones((L,), jnp.float32)
    for i in range(CH // L):
        ones_v[pl.ds(i * L, L)] = ones
        zeros_v[pl.ds(i * L, L)] = zeros

    def _zrow(r, _):
        for c in range(HH // L):
            rows_v[0, r, pl.ds(c * L, L)] = zeros
        return 0
    lax.fori_loop(0, CH, _zrow, 0)
    for k in range(RPT // CH):
        pltpu.sync_copy(rows_v.at[0],
                        acc_sh.at[pl.ds(sid * RPT + k * CH, CH)])
        if with_deg:
            pltpu.sync_copy(zeros_v,
                            deg_sh.at[pl.ds(sid * RPT + k * CH, CH)])
    plsc.subcore_barrier()

    def _run(y_c, do_deg):
        for g in range(GA):
            pltpu.async_copy(y_c.at[sidx_v.at[g]], rows_v.at[g], gsem)

        def _step(j, _):
            s = lax.rem(j, KR)
            p = lax.rem(j + GA, KR)

            @pl.when(j + GA >= KR)
            def _():
                pltpu.make_async_copy(rows_v.at[0], acc_sh.at[didx_v.at[0]],
                                      ssem).wait()

            pltpu.make_async_copy(y_c.at[sidx_v.at[0]], rows_v.at[0],
                                  gsem).wait()

            @pl.when(j + GA < NCH_E)
            def _():
                pltpu.async_copy(y_c.at[sidx_v.at[j + GA]], rows_v.at[p],
                                 gsem)

            pltpu.async_copy(rows_v.at[s], acc_sh.at[didx_v.at[j]], ssem,
                             add=True)
            if do_deg:
                @pl.when(j >= 1)
                def _():
                    pltpu.make_async_copy(ones_v, deg_sh.at[didx_v.at[0]],
                                          dsem).wait()
                pltpu.async_copy(ones_v, deg_sh.at[didx_v.at[j]], dsem,
                                 add=True)
            return 0

        lax.fori_loop(0, NCH_E, _step, 0)
        for _ in range(KR - GA):
            pltpu.make_async_copy(rows_v.at[0], acc_sh.at[didx_v.at[0]],
                                  ssem).wait()
        if do_deg:
            pltpu.make_async_copy(ones_v, deg_sh.at[didx_v.at[0]],
                                  dsem).wait()

    @pl.when(cid == 0)
    def _():
        _run(y_hbm.at[0], with_deg)

    @pl.when(cid == 1)
    def _():
        _run(y_hbm.at[1], False)

    plsc.subcore_barrier()

    for k in range(RPT // CH):
        r0 = sid * RPT + k * CH
        pltpu.sync_copy(acc_sh.at[pl.ds(r0, CH)],
                        agg_out.at[cid, pl.ds(r0, CH)])

    if with_deg:
        @pl.when(cid == 0)
        def _():
            pltpu.sync_copy(deg_sh.at[pl.ds(sid * RPT, RPT)],
                            deg_out.at[pl.ds(sid * RPT, RPT)])

  return _sc_aggregate


_sc_aggregate_deg = _make_sc_aggregate(True)
_sc_aggregate_nodeg = _make_sc_aggregate(False)


@functools.partial(
    pl.kernel,
    out_type=jax.ShapeDtypeStruct((2, MZ, H), jnp.float32),
    mesh=plsc.VectorSubcoreMesh(**_MESH),
    scratch_types=[
        pltpu.VMEM((NCH_Z, CHD), jnp.int32),
        pltpu.VMEM((NCH_Z, CHD), jnp.int32),
        pltpu.VMEM((KD, CHD, H), jnp.float32),
        pltpu.VMEM((KD, CHD, H), jnp.float32),
        pltpu.SemaphoreType.DMA,
        pltpu.SemaphoreType.DMA,
    ],
)
def _sc_decode_gather(h_hbm, aidx_hbm, bidx_hbm, z_out,
                      aidx_v, bidx_v, ra_v, rb_v, gsem, wsem):
    cid = lax.axis_index("c")
    sid = lax.axis_index("s")
    wid = sid * NC + cid
    pltpu.sync_copy(aidx_hbm.at[wid], aidx_v)
    pltpu.sync_copy(bidx_hbm.at[wid], bidx_v)

    base = wid * NCH_Z * CHD

    for g in range(GD2):
        pltpu.async_copy(h_hbm.at[aidx_v.at[g]], ra_v.at[g], gsem)
        pltpu.async_copy(h_hbm.at[bidx_v.at[g]], rb_v.at[g], gsem)

    def _step(j, _):
        s = lax.rem(j, KD)
        p = lax.rem(j + GD2, KD)

        @pl.when(j + GD2 >= KD)
        def _():
            pltpu.make_async_copy(ra_v.at[0], z_out.at[0, pl.ds(0, CHD)],
                                  wsem).wait()
            pltpu.make_async_copy(ra_v.at[0], z_out.at[0, pl.ds(0, CHD)],
                                  wsem).wait()

        pltpu.make_async_copy(h_hbm.at[aidx_v.at[0]], ra_v.at[0], gsem).wait()
        pltpu.make_async_copy(h_hbm.at[bidx_v.at[0]], rb_v.at[0], gsem).wait()

        @pl.when(j + GD2 < NCH_Z)
        def _():
            pltpu.async_copy(h_hbm.at[aidx_v.at[j + GD2]], ra_v.at[p], gsem)
            pltpu.async_copy(h_hbm.at[bidx_v.at[j + GD2]], rb_v.at[p], gsem)

        pltpu.async_copy(ra_v.at[s], z_out.at[0, pl.ds(base + j * CHD, CHD)],
                         wsem)
        pltpu.async_copy(rb_v.at[s], z_out.at[1, pl.ds(base + j * CHD, CHD)],
                         wsem)
        return 0

    lax.fori_loop(0, NCH_Z, _step, 0)
    for _ in range(2 * (KD - GD2)):
        pltpu.make_async_copy(ra_v.at[0], z_out.at[0, pl.ds(0, CHD)],
                              wsem).wait()


_BLK = 1024


def _tc_enc1_body(x_ref, ws_ref, wn_ref, b_ref, xs_ref, y_ref):
    xb = x_ref[...]
    xs_ref[...] = jnp.dot(xb, ws_ref[...],
                          preferred_element_type=jnp.float32) + b_ref[...]
    y = jnp.dot(xb, wn_ref[...], preferred_element_type=jnp.float32)
    y_ref[0] = y[:, :HH]
    y_ref[1] = y[:, HH:]


def _tc_enc1(xp, Ws1, Wn1, b1):
    return pl.pallas_call(
        _tc_enc1_body,
        grid=(NP // _BLK,),
        in_specs=[
            pl.BlockSpec((_BLK, D), lambda i: (i, 0)),
            pl.BlockSpec((D, H), lambda i: (0, 0)),
            pl.BlockSpec((D, H), lambda i: (0, 0)),
            pl.BlockSpec((1, H), lambda i: (0, 0)),
        ],
        out_specs=[
            pl.BlockSpec((_BLK, H), lambda i: (i, 0)),
            pl.BlockSpec((NC, _BLK, HH), lambda i: (0, i, 0)),
        ],
        out_shape=[
            jax.ShapeDtypeStruct((NP, H), jnp.float32),
            jax.ShapeDtypeStruct((NC, NP, HH), jnp.float32),
        ],
    )(xp, Ws1, Wn1, b1.reshape(1, H))


def _mean_from_parts(agg_ref, deg_ref):
    agg = jnp.concatenate([agg_ref[0], agg_ref[1]], axis=1)
    return agg * (1.0 / jnp.maximum(deg_ref[...], 1.0))


def _tc_enc2_body(xs_ref, agg_ref, deg_ref, ws_ref, wn_ref, b_ref,
                  hs_ref, y_ref):
    h1 = jnp.maximum(xs_ref[...] + _mean_from_parts(agg_ref, deg_ref), 0.0)
    hs_ref[...] = jnp.dot(h1, ws_ref[...],
                          preferred_element_type=jnp.float32) + b_ref[...]
    y = jnp.dot(h1, wn_ref[...], preferred_element_type=jnp.float32)
    y_ref[0] = y[:, :HH]
    y_ref[1] = y[:, HH:]


def _tc_enc2(xs1, agg1, deg1, Ws2, Wn2, b2):
    return pl.pallas_call(
        _tc_enc2_body,
        grid=(NP // _BLK,),
        in_specs=[
            pl.BlockSpec((_BLK, H), lambda i: (i, 0)),
            pl.BlockSpec((NC, _BLK, HH), lambda i: (0, i, 0)),
            pl.BlockSpec((_BLK, 1), lambda i: (i, 0)),
            pl.BlockSpec((H, H), lambda i: (0, 0)),
            pl.BlockSpec((H, H), lambda i: (0, 0)),
            pl.BlockSpec((1, H), lambda i: (0, 0)),
        ],
        out_specs=[
            pl.BlockSpec((_BLK, H), lambda i: (i, 0)),
            pl.BlockSpec((NC, _BLK, HH), lambda i: (0, i, 0)),
        ],
        out_shape=[
            jax.ShapeDtypeStruct((NP, H), jnp.float32),
            jax.ShapeDtypeStruct((NC, NP, HH), jnp.float32),
        ],
    )(xs1, agg1, deg1.reshape(NP, 1), Ws2, Wn2, b2.reshape(1, H))


def _tc_combine_body(hs_ref, agg_ref, deg_ref, h_ref):
    h_ref[...] = hs_ref[...] + _mean_from_parts(agg_ref, deg_ref)


def _tc_combine(hs2, agg2, deg1):
    return pl.pallas_call(
        _tc_combine_body,
        grid=(NP // _BLK,),
        in_specs=[
            pl.BlockSpec((_BLK, H), lambda i: (i, 0)),
            pl.BlockSpec((NC, _BLK, HH), lambda i: (0, i, 0)),
            pl.BlockSpec((_BLK, 1), lambda i: (i, 0)),
        ],
        out_specs=pl.BlockSpec((_BLK, H), lambda i: (i, 0)),
        out_shape=jax.ShapeDtypeStruct((NP, H), jnp.float32),
    )(hs2, agg2, deg1.reshape(NP, 1))


_BLKZ = 2048


def _tc_mlp_body(z_ref, w1_ref, b1_ref, w2_ref, b2_ref, w3_ref, b3_ref,
                 o_ref):
    z = z_ref[0] * z_ref[1]
    z = jnp.maximum(jnp.dot(z, w1_ref[...],
                            preferred_element_type=jnp.float32)
                    + b1_ref[...], 0.0)
    z = jnp.maximum(jnp.dot(z, w2_ref[...],
                            preferred_element_type=jnp.float32)
                    + b2_ref[...], 0.0)
    o_ref[...] = jnp.dot(z, w3_ref[...],
                         preferred_element_type=jnp.float32) + b3_ref[...]


def _tc_mlp(z, Wd1, bd1, Wd2, bd2, Wd3, bd3):
    return pl.pallas_call(
        _tc_mlp_body,
        grid=(MZ // _BLKZ,),
        in_specs=[
            pl.BlockSpec((2, _BLKZ, H), lambda i: (0, i, 0)),
            pl.BlockSpec((H, H), lambda i: (0, 0)),
            pl.BlockSpec((1, H), lambda i: (0, 0)),
            pl.BlockSpec((H, H), lambda i: (0, 0)),
            pl.BlockSpec((1, H), lambda i: (0, 0)),
            pl.BlockSpec((H, 1), lambda i: (0, 0)),
            pl.BlockSpec((1, 1), lambda i: (0, 0)),
        ],
        out_specs=pl.BlockSpec((_BLKZ, 1), lambda i: (i, 0)),
        out_shape=jax.ShapeDtypeStruct((MZ, 1), jnp.float32),
    )(z, Wd1, bd1.reshape(1, H), Wd2, bd2.reshape(1, H), Wd3,
      bd3.reshape(1, 1))


def kernel(x, edge_index, pos_edge_index, neg_edge_index, Ws1, Wn1, b1,
           Ws2, Wn2, b2, Wd1, bd1, Wd2, bd2, Wd3, bd3):
    xp = jnp.pad(x, ((0, NP - N), (0, 0)))

    pad_e = jnp.full((EPAD - E,), NP - 1, jnp.int32)
    srcp = jnp.concatenate([edge_index[0], pad_e]).reshape(NS, NCH_E, CH)
    dstp = jnp.concatenate([edge_index[1], pad_e]).reshape(NS, NCH_E, CH)

    pad_z = jnp.zeros((MZ - MZ_RAW,), jnp.int32)
    aidx = jnp.concatenate(
        [pos_edge_index[0], neg_edge_index[0], pad_z]).reshape(NW, NCH_Z, CHD)
    bidx = jnp.concatenate(
        [pos_edge_index[1], neg_edge_index[1], pad_z]).reshape(NW, NCH_Z, CHD)
    srcp, dstp, aidx, bidx = lax.optimization_barrier((srcp, dstp, aidx, bidx))

    xs1, y1 = _tc_enc1(xp, Ws1, Wn1, b1)
    agg1, deg1 = _sc_aggregate_deg(y1, srcp, dstp)
    hs2, y2 = _tc_enc2(xs1, agg1, deg1, Ws2, Wn2, b2)
    agg2, = _sc_aggregate_nodeg(y2, srcp, dstp)
    h = _tc_combine(hs2, agg2, deg1)
    z = _sc_decode_gather(h, aidx, bidx)
    scores = _tc_mlp(z, Wd1, bd1, Wd2, bd2, Wd3, bd3)
    return (scores[:EP], scores[EP:2 * EP])
